# baseline (speedup 1.0000x reference)
import jax
import jax.numpy as jnp
from jax import lax
from jax.experimental import pallas as pl
from jax.experimental.pallas import tpu as pltpu
from jax.experimental.pallas import tpu_sc as plsc

_BS, _N, _D = 16, 16384, 128

_NSC = 4096
_LH = _NSC // 2
_W = 128

_TBLK = 4096
_NTB = (_N - _NSC) // _TBLK
_TOFF = _NSC // _TBLK


def _sc_body(pred_hbm, target_hbm, mask_hbm, out_hbm,
             mask_v, idx_v, rows_p, rows_t, acc_v, sem_p, sem_t):
    c = lax.axis_index("c")
    s = lax.axis_index("s")
    wid = s * 2 + c
    b = wid // 2
    h = wid % 2
    start = b * _N + h * _LH

    pltpu.sync_copy(mask_hbm.at[pl.ds(start, _LH)], mask_v)

    safe = jnp.full((16,), start, jnp.int32)

    def prefill(i, carry):
        idx_v[pl.ds(i * 16, 16)] = safe
        return carry

    lax.fori_loop(0, _LH // 16, prefill, 0)

    def compact(i, off):
        mi = mask_v[pl.ds(i * 16, 16)]
        keep = mi != 0
        one = keep.astype(jnp.int32)
        csum = plsc.cumsum(one)
        pos = csum + (off - 1)
        rowids = lax.iota(jnp.int32, 16) + (start + i * 16)
        plsc.store_scatter(idx_v, [pos], rowids, mask=keep)
        return off + jnp.sum(one)

    count = lax.fori_loop(0, _LH // 16, compact, jnp.int32(0))

    nwin = (count + _W - 1) // _W

    def window(w, accs):
        off = w * _W
        idx_slice = idx_v.at[pl.ds(off, _W)]
        cp = pltpu.async_copy(pred_hbm.at[idx_slice], rows_p, sem_p)
        ct = pltpu.async_copy(target_hbm.at[idx_slice], rows_t, sem_t)
        cp.wait()
        ct.wait()
        nrows = jnp.minimum(count - off, _W)

        def row(r, a):
            a1, a2, a3 = a
            for ch in range(_D // 16):
                pch = rows_p[r, pl.ds(ch * 16, 16)]
                tch = rows_t[r, pl.ds(ch * 16, 16)]
                a1 = a1 + pch * tch
                a2 = a2 + pch * pch
                a3 = a3 + tch * tch
            return (a1, a2, a3)

        return lax.fori_loop(0, nrows, row, accs)

    zero = jnp.zeros((16,), jnp.float32)
    a1, a2, a3 = lax.fori_loop(0, nwin, window, (zero, zero, zero))

    acc_v[pl.ds(0, 16)] = a1
    acc_v[pl.ds(16, 16)] = a2
    acc_v[pl.ds(32, 16)] = a3
    acc_v[pl.ds(48, 16)] = jnp.full((16,), 1.0, jnp.float32) * count.astype(jnp.float32)
    pltpu.sync_copy(acc_v, out_hbm.at[wid])


def _sc_call(pred_flat, target_flat, mask_flat):
    mesh = plsc.VectorSubcoreMesh(core_axis_name="c", subcore_axis_name="s")
    kern = pl.kernel(
        _sc_body,
        out_type=jax.ShapeDtypeStruct((32, 64), jnp.float32),
        mesh=mesh,
        scratch_types=[
            pltpu.VMEM((_LH,), jnp.int32),
            pltpu.VMEM((_LH,), jnp.int32),
            pltpu.VMEM((_W, _D), jnp.float32),
            pltpu.VMEM((_W, _D), jnp.float32),
            pltpu.VMEM((64,), jnp.float32),
            pltpu.SemaphoreType.DMA,
            pltpu.SemaphoreType.DMA,
        ],
        compiler_params=pltpu.CompilerParams(needs_layout_passes=False),
    )
    return kern(pred_flat, target_flat, mask_flat)


def _tc_body(mask_ref, pred_ref, target_ref, out_ref, acc_ref, cnt_ref):
    b = pl.program_id(0)
    i = pl.program_id(1)

    @pl.when(i == 0)
    def _():
        acc_ref[...] = jnp.zeros_like(acc_ref)
        cnt_ref[0] = 0.0

    m = (mask_ref[0, 0, :] != 0).astype(jnp.float32)
    mf = m[:, None]
    p = pred_ref[0]
    t = target_ref[0]
    mp = (p * mf).reshape(_TBLK // 8, 8, _D)
    mt = (t * mf).reshape(_TBLK // 8, 8, _D)
    pr = p.reshape(_TBLK // 8, 8, _D)
    tr = t.reshape(_TBLK // 8, 8, _D)
    acc_ref[0] += jnp.sum(mp * tr, axis=0)
    acc_ref[1] += jnp.sum(mp * pr, axis=0)
    acc_ref[2] += jnp.sum(mt * tr, axis=0)
    cnt_ref[0] += jnp.sum(m)

    @pl.when(i == _NTB - 1)
    def _():
        out_ref[b, 0] = jnp.sum(acc_ref[0])
        out_ref[b, 1] = jnp.sum(acc_ref[1])
        out_ref[b, 2] = jnp.sum(acc_ref[2])
        out_ref[b, 3] = cnt_ref[0]


def _tc_call(mask3, pred, target):
    return pl.pallas_call(
        _tc_body,
        grid=(_BS, _NTB),
        in_specs=[
            pl.BlockSpec((1, 1, _TBLK), lambda b, i: (b * (_N // _TBLK) + _TOFF + i, 0, 0)),
            pl.BlockSpec((1, _TBLK, _D), lambda b, i: (b, _TOFF + i, 0)),
            pl.BlockSpec((1, _TBLK, _D), lambda b, i: (b, _TOFF + i, 0)),
        ],
        out_specs=pl.BlockSpec(memory_space=pltpu.SMEM),
        out_shape=jax.ShapeDtypeStruct((_BS, 4), jnp.float32),
        scratch_shapes=[pltpu.VMEM((3, 8, _D), jnp.float32),
                        pltpu.SMEM((1,), jnp.float32)],
    )(mask3, pred, target)


def kernel(pred, target, mask):
    pred_flat = pred.reshape(_BS * _N, _D)
    target_flat = target.reshape(_BS * _N, _D)
    mask_flat = mask.reshape(_BS * _N)
    mask3 = mask.reshape(_BS * (_N // _TBLK), 1, _TBLK)

    sc = _sc_call(pred_flat, target_flat, mask_flat)
    tc = _tc_call(mask3, pred, target)

    scr = sc.reshape(_BS, 2, 4, 16)
    dot = tc[:, 0] + jnp.sum(scr[:, :, 0, :], axis=(1, 2))
    pp = tc[:, 1] + jnp.sum(scr[:, :, 1, :], axis=(1, 2))
    tt = tc[:, 2] + jnp.sum(scr[:, :, 2, :], axis=(1, 2))
    cnt = tc[:, 3] + scr[:, 0, 3, 0] + scr[:, 1, 3, 0]

    denom = jnp.sqrt(pp) * jnp.sqrt(tt)
    safe = jnp.where(denom > 0.0, denom, 1.0)
    losses = jnp.where(cnt > 0.0, -dot / safe, 0.0)
    return jnp.sum(losses) / _BS
